# Initial kernel scaffold; baseline (speedup 1.0000x reference)
import jax
import jax.numpy as jnp
from jax.experimental import pallas as pl


def kernel(h_V, h_P, P_idx, batch_id, params):
    raise NotImplementedError("write your pallas kernel here")


# speedup vs baseline: 11.9080x; 11.9080x over previous
import functools

import jax
import jax.numpy as jnp
from jax import lax
from jax.experimental import pallas as pl
from jax.experimental.pallas import tpu as pltpu
from jax.experimental.pallas import tpu_sc as plsc

N = 10000
E = 320000
H = 128
HEADS = 4
DH = H // HEADS
G = 16
NC, NS = 2, 16
NW = NC * NS
CHUNK = 80
GIT = (2 * E) // NW // CHUNK
SIT = E // NW // CHUNK
EPW = E // NW
NPAD = 10240
NPT = NPAD // NS
BE = 1280
NEB = E // BE
ISD = float(DH) ** -0.5

_MESH = plsc.VectorSubcoreMesh(core_axis_name="c", subcore_axis_name="s")


@functools.partial(
    pl.kernel,
    out_type=jax.ShapeDtypeStruct((2 * E, H), jnp.float32),
    mesh=_MESH,
    scratch_types=[
        pltpu.VMEM((GIT, CHUNK), jnp.int32),
        pltpu.VMEM((CHUNK, H), jnp.float32),
        pltpu.SemaphoreType.DMA,
    ],
)
def _sc_gather(table, idx3, out, idx_v, buf, sem):
    wid = lax.axis_index("s") * NC + lax.axis_index("c")
    base = wid * (GIT * CHUNK)
    pltpu.sync_copy(idx3.at[wid], idx_v)

    def body(k, carry):
        pltpu.async_copy(table.at[idx_v.at[k]], buf, sem).wait()
        pltpu.sync_copy(buf, out.at[pl.ds(base + k * CHUNK, CHUNK)])
        return carry

    lax.fori_loop(0, GIT, body, 0)


def _make_sc_scatter(w):
    @functools.partial(
        pl.kernel,
        out_type=jax.ShapeDtypeStruct((NC * NPAD, w), jnp.float32),
        mesh=_MESH,
        scratch_types=[
            pltpu.VMEM((SIT, CHUNK), jnp.int32),
            pltpu.VMEM((CHUNK, w), jnp.float32),
            pltpu.VMEM_SHARED((NPAD, w), jnp.float32),
        ],
    )
    def scat(vals, src3, z_v, out, idx_v, vbuf, acc):
        cid = lax.axis_index("c")
        sid = lax.axis_index("s")
        wid = sid * NC + cid
        pltpu.sync_copy(z_v.at[pl.ds(sid * NPT, NPT)],
                        acc.at[pl.ds(sid * NPT, NPT)])
        plsc.subcore_barrier()

        base = wid * EPW
        pltpu.sync_copy(src3.at[wid], idx_v)

        def body(k, carry):
            off = base + k * CHUNK
            pltpu.sync_copy(vals.at[pl.ds(off, CHUNK)], vbuf)
            pltpu.sync_copy(vbuf, acc.at[idx_v.at[k]], add=True)
            return carry

        lax.fori_loop(0, SIT, body, 0)
        plsc.subcore_barrier()
        pltpu.sync_copy(acc.at[pl.ds(sid * NPT, NPT)],
                        out.at[pl.ds(cid * NPAD + sid * NPT, NPT)])

    return scat


_sc_scatter_v = _make_sc_scatter(H)


def _head_expand():
    lane_head = lax.broadcasted_iota(jnp.int32, (HEADS, H), 1) // DH
    row = lax.broadcasted_iota(jnp.int32, (HEADS, H), 0)
    return (lane_head == row).astype(jnp.float32)


def _graph_onehot(ids, rows):
    return (ids[...] == lax.broadcasted_iota(jnp.int32, (rows, G), 1)
            ).astype(jnp.float32)


def _attn_body(hs, hp, hd, bs, bd, gg, wb1s, wb1p, wb1d, bb1, wb2, bb2,
               wb3, bb3, wv1p, wv1d, bv1, wv2, bv2, wv3, bv3,
               lg_o, v_o, gm_o):
    i = pl.program_id(0)
    gg_ = gg[...]
    hs_ = hs[...] * (_graph_onehot(bs, BE) @ gg_)
    hd_ = hd[...] * (_graph_onehot(bd, BE) @ gg_)
    hp_ = hp[...]
    t = jax.nn.gelu(hs_ @ wb1s[...] + hp_ @ wb1p[...] + hd_ @ wb1d[...] + bb1[...])
    t = jax.nn.gelu(t @ wb2[...] + bb2[...])
    lg = t @ wb3[...] + bb3[...]
    u = jax.nn.gelu(hp_ @ wv1p[...] + hd_ @ wv1d[...] + bv1[...])
    u = jax.nn.gelu(u @ wv2[...] + bv2[...])
    v = u @ wv3[...] + bv3[...]
    lg_o[...] = lg
    v_o[...] = v
    m = jnp.max(lg)

    @pl.when(i == 0)
    def _():
        gm_o[...] = jnp.full((8, 128), -3e38, jnp.float32)

    gm_o[...] = jnp.maximum(gm_o[...], m)


def _edge_attn(hs, hp, hd, bsrc, bdst, gg, p):
    eb = lambda w: pl.BlockSpec((BE, w), lambda i: (i, 0))
    cb = lambda a: pl.BlockSpec(a.shape, lambda i: (0,) * a.ndim)
    wb1s, wb1p, wb1d = p['B1'][0][:H], p['B1'][0][H:2 * H], p['B1'][0][2 * H:]
    wv1p, wv1d = p['V1'][0][:H], p['V1'][0][H:]
    args = (hs, hp, hd, bsrc, bdst, gg,
            wb1s, wb1p, wb1d, p['B1'][1].reshape(1, H),
            p['B2'][0], p['B2'][1].reshape(1, H),
            p['B3'][0], p['B3'][1].reshape(1, HEADS),
            wv1p, wv1d, p['V1'][1].reshape(1, H),
            p['V2'][0], p['V2'][1].reshape(1, H),
            p['V3'][0], p['V3'][1].reshape(1, H))
    return pl.pallas_call(
        _attn_body,
        grid=(NEB,),
        in_specs=[eb(H), eb(H), eb(H), eb(1), eb(1), cb(gg)]
                 + [cb(a) for a in args[6:]],
        out_specs=[eb(HEADS), eb(H), pl.BlockSpec((8, 128), lambda i: (0, 0))],
        out_shape=[jax.ShapeDtypeStruct((E, HEADS), jnp.float32),
                   jax.ShapeDtypeStruct((E, H), jnp.float32),
                   jax.ShapeDtypeStruct((8, 128), jnp.float32)],
    )(*args)


def _weight_body(lg, v, gm, p_o, e_o):
    r4 = _head_expand()
    eb = jnp.exp((lg[...] - gm[0:1, 0:1]) * ISD) @ r4
    e_o[...] = eb
    p_o[...] = eb * v[...]


def _edge_weight(lg, v, gm):
    eb = lambda w: pl.BlockSpec((BE, w), lambda i: (i, 0))
    return pl.pallas_call(
        _weight_body,
        grid=(NEB,),
        in_specs=[eb(HEADS), eb(H), pl.BlockSpec((8, 128), lambda i: (0, 0))],
        out_specs=[eb(H), eb(H)],
        out_shape=[jax.ShapeDtypeStruct((E, H), jnp.float32),
                   jax.ShapeDtypeStruct((E, H), jnp.float32)],
    )(lg, v, gm)


def _node_body_common(hv, ggp, av0, av1, as0, as1, bid, wo, g0, b0, f1w, f1b,
                      f2w, f2b, g1, b1, wg1, bg1, wg2, bg2, wg3, bg3):
    oh = _graph_onehot(bid, N)
    hvg = hv[...] * (oh @ ggp[...])
    sden = as0[...] + as1[...]
    agg = av0[...] + av1[...]
    agg = jnp.where(sden > 0.0, agg / sden, 0.0)
    x = hvg + agg @ wo[...]
    mu = jnp.mean(x, axis=0, keepdims=True)
    xc = x - mu
    var = jnp.mean(xc * xc, axis=0, keepdims=True)
    x = xc * lax.rsqrt(var + 1e-5) * g0[...] + b0[...]
    f = f2b[...]
    for cix in range(4):
        hid = jnp.maximum(x @ f1w[:, cix * H:(cix + 1) * H]
                          + f1b[:, cix * H:(cix + 1) * H], 0.0)
        f = f + hid @ f2w[cix * H:(cix + 1) * H, :]
    y = x + f
    mu2 = jnp.mean(y, axis=0, keepdims=True)
    yc = y - mu2
    var2 = jnp.mean(yc * yc, axis=0, keepdims=True)
    h_pre = yc * lax.rsqrt(var2 + 1e-5) * g1[...] + b1[...]
    c = lax.dot_general(oh, h_pre, (((0,), (0,)), ((), ())))
    cnt = lax.dot_general(oh, jnp.ones((N, 1), jnp.float32),
                          (((0,), (0,)), ((), ())))
    c = c / jnp.maximum(cnt, 1.0)
    gg = jnp.maximum(c @ wg1[...] + bg1[...], 0.0)
    gg = jnp.maximum(gg @ wg2[...] + bg2[...], 0.0)
    gg = jax.nn.sigmoid(gg @ wg3[...] + bg3[...])
    return h_pre, gg, oh


def _node_body_mid(*refs):
    h_o, gg_o = refs[-2], refs[-1]
    h_pre, gg, _ = _node_body_common(*refs[:-2])
    h_o[...] = h_pre
    gg_o[...] = gg


def _node_body_last(*refs):
    wr, br, out = refs[-3], refs[-2], refs[-1]
    h_pre, gg, oh = _node_body_common(*refs[:-3])
    hv = h_pre * (oh @ gg)
    lgt = hv @ wr[...] + br[...]
    mx = jnp.max(lgt, axis=1, keepdims=True)
    sh = lgt - mx
    out[...] = sh - jnp.log(jnp.sum(jnp.exp(sh), axis=1, keepdims=True))


def _node(hv, ggp, av0, av1, as0, as1, bid2, p, readout):
    args = [hv, ggp, av0, av1, as0, as1, bid2,
            p['WO'], p['bn0'][0].reshape(1, H), p['bn0'][1].reshape(1, H),
            p['F1'][0], p['F1'][1].reshape(1, 4 * H),
            p['F2'][0], p['F2'][1].reshape(1, H),
            p['bn1'][0].reshape(1, H), p['bn1'][1].reshape(1, H),
            p['G1'][0], p['G1'][1].reshape(1, H),
            p['G2'][0], p['G2'][1].reshape(1, H),
            p['G3'][0], p['G3'][1].reshape(1, H)]
    if readout is None:
        body = _node_body_mid
        out_shape = [jax.ShapeDtypeStruct((N, H), jnp.float32),
                     jax.ShapeDtypeStruct((G, H), jnp.float32)]
    else:
        args += [readout[0], readout[1].reshape(1, 20)]
        body = _node_body_last
        out_shape = jax.ShapeDtypeStruct((N, 20), jnp.float32)
    return pl.pallas_call(body, out_shape=out_shape)(*args)


def _eupd_body(hs, hp, hd, we1s, we1p, we1d, be1, we2, be2, we3, be3,
               y_o, s1_o, s2_o):
    i = pl.program_id(0)
    m = jax.nn.gelu(hs[...] @ we1s[...] + hp[...] @ we1p[...]
                    + hd[...] @ we1d[...] + be1[...])
    m = jax.nn.gelu(m @ we2[...] + be2[...])
    m = m @ we3[...] + be3[...]
    y = hp[...] + m
    y_o[...] = y

    @pl.when(i == 0)
    def _():
        s1_o[...] = jnp.zeros((8, 128), jnp.float32)
        s2_o[...] = jnp.zeros((8, 128), jnp.float32)

    s1_o[...] = s1_o[...] + jnp.sum(y, axis=0, keepdims=True)
    s2_o[...] = s2_o[...] + jnp.sum(y * y, axis=0, keepdims=True)


def _edge_update(hs, hp, hd, p):
    eb = lambda w: pl.BlockSpec((BE, w), lambda i: (i, 0))
    cb = lambda a: pl.BlockSpec(a.shape, lambda i: (0,) * a.ndim)
    we1s, we1p, we1d = p['E1'][0][:H], p['E1'][0][H:2 * H], p['E1'][0][2 * H:]
    args = (hs, hp, hd,
            we1s, we1p, we1d, p['E1'][1].reshape(1, H),
            p['E2'][0], p['E2'][1].reshape(1, H),
            p['E3'][0], p['E3'][1].reshape(1, H))
    acc = pl.BlockSpec((8, 128), lambda i: (0, 0))
    return pl.pallas_call(
        _eupd_body,
        grid=(NEB,),
        in_specs=[eb(H), eb(H), eb(H)] + [cb(a) for a in args[3:]],
        out_specs=[eb(H), acc, acc],
        out_shape=[jax.ShapeDtypeStruct((E, H), jnp.float32),
                   jax.ShapeDtypeStruct((8, 128), jnp.float32),
                   jax.ShapeDtypeStruct((8, 128), jnp.float32)],
    )(*args)


def _enorm_body(y, s1, s2, gma, bta, o):
    mu = s1[0:1, :] * (1.0 / E)
    var = s2[0:1, :] * (1.0 / E) - mu * mu
    o[...] = (y[...] - mu) * lax.rsqrt(var + 1e-5) * gma[...] + bta[...]


def _edge_norm(y, s1, s2, gb):
    eb = pl.BlockSpec((BE, H), lambda i: (i, 0))
    cb = pl.BlockSpec((8, 128), lambda i: (0, 0))
    bb = pl.BlockSpec((1, H), lambda i: (0, 0))
    return pl.pallas_call(
        _enorm_body,
        grid=(NEB,),
        in_specs=[eb, cb, cb, bb, bb],
        out_specs=eb,
        out_shape=jax.ShapeDtypeStruct((E, H), jnp.float32),
    )(y, s1, s2, gb[0].reshape(1, H), gb[1].reshape(1, H))


def kernel(h_V, h_P, P_idx, batch_id, params):
    idx3g = P_idx.reshape(NW, GIT, CHUNK)
    src3 = P_idx[0].reshape(NW, SIT, CHUNK)
    bid2 = batch_id.reshape(N, 1)
    bsrc = batch_id[P_idx[0]].reshape(E, 1)
    bdst = batch_id[P_idx[1]].reshape(E, 1)
    z_v = jnp.zeros((NPAD, H), jnp.float32)

    hp = h_P
    hv = h_V
    gg = jnp.ones((G, H), jnp.float32)
    g = _sc_gather(hv, idx3g)
    hs, hd = g[:E], g[E:]
    layers = params['layers']
    for l, p in enumerate(layers):
        lg, v, gm = _edge_attn(hs, hp, hd, bsrc, bdst, gg, p)
        p_arr, ew = _edge_weight(lg, v, gm)
        acc_v = _sc_scatter_v(p_arr, src3, z_v)
        acc_s = _sc_scatter_v(ew, src3, z_v)
        last = l == len(layers) - 1
        res = _node(hv, gg, acc_v[:N], acc_v[NPAD:NPAD + N], acc_s[:N],
                    acc_s[NPAD:NPAD + N], bid2, p,
                    params['readout'] if last else None)
        if last:
            return res
        hv, gg = res
        g = _sc_gather(hv, idx3g)
        hs, hd = g[:E], g[E:]
        y, s1, s2 = _edge_update(hs, hp, hd, p)
        hp = _edge_norm(y, s1, s2, p['bne'])


# speedup vs baseline: 12.3442x; 1.0366x over previous
import functools

import jax
import jax.numpy as jnp
from jax import lax
from jax.experimental import pallas as pl
from jax.experimental.pallas import tpu as pltpu
from jax.experimental.pallas import tpu_sc as plsc

N = 10000
E = 320000
H = 128
HEADS = 4
DH = H // HEADS
G = 16
NC, NS = 2, 16
NW = NC * NS
CHUNK = 80
GIT = (2 * E) // NW // CHUNK
SIT = E // NW // CHUNK
EPW = E // NW
NPAD = 10240
NPT = NPAD // NS
BE = 1280
NEB = E // BE
ISD = float(DH) ** -0.5

_MESH = plsc.VectorSubcoreMesh(core_axis_name="c", subcore_axis_name="s")


@functools.partial(
    pl.kernel,
    out_type=jax.ShapeDtypeStruct((2 * E, H), jnp.float32),
    mesh=_MESH,
    scratch_types=[
        pltpu.VMEM((GIT, CHUNK), jnp.int32),
        pltpu.VMEM((CHUNK, H), jnp.float32),
        pltpu.SemaphoreType.DMA,
    ],
)
def _sc_gather(table, idx3, out, idx_v, buf, sem):
    wid = lax.axis_index("s") * NC + lax.axis_index("c")
    base = wid * (GIT * CHUNK)
    pltpu.sync_copy(idx3.at[wid], idx_v)

    def body(k, carry):
        pltpu.async_copy(table.at[idx_v.at[k]], buf, sem).wait()
        pltpu.sync_copy(buf, out.at[pl.ds(base + k * CHUNK, CHUNK)])
        return carry

    lax.fori_loop(0, GIT, body, 0)


def _make_sc_scatter(w):
    @functools.partial(
        pl.kernel,
        out_type=jax.ShapeDtypeStruct((NC * NPAD, w), jnp.float32),
        mesh=_MESH,
        scratch_types=[
            pltpu.VMEM((SIT, CHUNK), jnp.int32),
            pltpu.VMEM((CHUNK, w), jnp.float32),
            pltpu.VMEM_SHARED((NPAD, w), jnp.float32),
        ],
    )
    def scat(vals, src3, z_v, out, idx_v, vbuf, acc):
        cid = lax.axis_index("c")
        sid = lax.axis_index("s")
        wid = sid * NC + cid
        pltpu.sync_copy(z_v.at[pl.ds(sid * NPT, NPT)],
                        acc.at[pl.ds(sid * NPT, NPT)])
        plsc.subcore_barrier()

        base = wid * EPW
        pltpu.sync_copy(src3.at[wid], idx_v)

        def body(k, carry):
            off = base + k * CHUNK
            pltpu.sync_copy(vals.at[pl.ds(off, CHUNK)], vbuf)
            pltpu.sync_copy(vbuf, acc.at[idx_v.at[k]], add=True)
            return carry

        lax.fori_loop(0, SIT, body, 0)
        plsc.subcore_barrier()
        pltpu.sync_copy(acc.at[pl.ds(sid * NPT, NPT)],
                        out.at[pl.ds(cid * NPAD + sid * NPT, NPT)])

    return scat


_sc_scatter_v = _make_sc_scatter(H)


def _mm(a, b):
    return lax.dot_general(a.astype(jnp.bfloat16), b.astype(jnp.bfloat16),
                           (((1,), (0,)), ((), ())),
                           preferred_element_type=jnp.float32)


def _head_expand():
    lane_head = lax.broadcasted_iota(jnp.int32, (HEADS, H), 1) // DH
    row = lax.broadcasted_iota(jnp.int32, (HEADS, H), 0)
    return (lane_head == row).astype(jnp.float32)


def _graph_onehot(ids, rows):
    return (ids[...] == lax.broadcasted_iota(jnp.int32, (rows, G), 1)
            ).astype(jnp.float32)


def _attn_body(hs, hp, hd, s1, s2, gma, bta, bs, bd, gg, wb1s, wb1p, wb1d,
               bb1, wb2, bb2, wb3, bb3, wv1p, wv1d, bv1, wv2, bv2, wv3, bv3,
               lg_o, v_o, gm_o):
    i = pl.program_id(0)
    gg_ = gg[...]
    hs_ = hs[...] * (_graph_onehot(bs, BE) @ gg_)
    hd_ = hd[...] * (_graph_onehot(bd, BE) @ gg_)
    mu = s1[0:1, :] * (1.0 / E)
    var = s2[0:1, :] * (1.0 / E) - mu * mu
    hp_ = (hp[...] - mu) * lax.rsqrt(var + 1e-5) * gma[...] + bta[...]
    t = jax.nn.gelu(_mm(hs_, wb1s[...]) + _mm(hp_, wb1p[...])
                    + _mm(hd_, wb1d[...]) + bb1[...])
    t = jax.nn.gelu(_mm(t, wb2[...]) + bb2[...])
    lg = _mm(t, wb3[...]) + bb3[...]
    u = jax.nn.gelu(_mm(hp_, wv1p[...]) + _mm(hd_, wv1d[...]) + bv1[...])
    u = jax.nn.gelu(_mm(u, wv2[...]) + bv2[...])
    v = _mm(u, wv3[...]) + bv3[...]
    lg_o[...] = lg
    v_o[...] = v
    m = jnp.max(lg)

    @pl.when(i == 0)
    def _():
        gm_o[...] = jnp.full((8, 128), -3e38, jnp.float32)

    gm_o[...] = jnp.maximum(gm_o[...], m)


def _edge_attn(hs, hp, hd, hpstats, bsrc, bdst, gg, p):
    eb = lambda w: pl.BlockSpec((BE, w), lambda i: (i, 0))
    cb = lambda a: pl.BlockSpec(a.shape, lambda i: (0,) * a.ndim)
    wb1s, wb1p, wb1d = p['B1'][0][:H], p['B1'][0][H:2 * H], p['B1'][0][2 * H:]
    wv1p, wv1d = p['V1'][0][:H], p['V1'][0][H:]
    s1, s2, gma, bta = hpstats
    args = (hs, hp, hd, s1, s2, gma, bta, bsrc, bdst, gg,
            wb1s, wb1p, wb1d, p['B1'][1].reshape(1, H),
            p['B2'][0], p['B2'][1].reshape(1, H),
            p['B3'][0], p['B3'][1].reshape(1, HEADS),
            wv1p, wv1d, p['V1'][1].reshape(1, H),
            p['V2'][0], p['V2'][1].reshape(1, H),
            p['V3'][0], p['V3'][1].reshape(1, H))
    sb = pl.BlockSpec((8, 128), lambda i: (0, 0))
    bb = pl.BlockSpec((1, H), lambda i: (0, 0))
    return pl.pallas_call(
        _attn_body,
        grid=(NEB,),
        in_specs=[eb(H), eb(H), eb(H), sb, sb, bb, bb, eb(1), eb(1), cb(gg)]
                 + [cb(a) for a in args[10:]],
        out_specs=[eb(HEADS), eb(H), pl.BlockSpec((8, 128), lambda i: (0, 0))],
        out_shape=[jax.ShapeDtypeStruct((E, HEADS), jnp.float32),
                   jax.ShapeDtypeStruct((E, H), jnp.float32),
                   jax.ShapeDtypeStruct((8, 128), jnp.float32)],
    )(*args)


def _weight_body(lg, v, gm, p_o, e_o):
    r4 = _head_expand()
    eb = jnp.exp((lg[...] - gm[0:1, 0:1]) * ISD) @ r4
    e_o[...] = eb
    p_o[...] = eb * v[...]


def _edge_weight(lg, v, gm):
    eb = lambda w: pl.BlockSpec((BE, w), lambda i: (i, 0))
    return pl.pallas_call(
        _weight_body,
        grid=(NEB,),
        in_specs=[eb(HEADS), eb(H), pl.BlockSpec((8, 128), lambda i: (0, 0))],
        out_specs=[eb(H), eb(H)],
        out_shape=[jax.ShapeDtypeStruct((E, H), jnp.float32),
                   jax.ShapeDtypeStruct((E, H), jnp.float32)],
    )(lg, v, gm)


def _node_body_common(hv, ggp, av0, av1, as0, as1, bid, wo, g0, b0, f1w, f1b,
                      f2w, f2b, g1, b1, wg1, bg1, wg2, bg2, wg3, bg3):
    oh = _graph_onehot(bid, N)
    hvg = hv[...] * (oh @ ggp[...])
    sden = as0[...] + as1[...]
    agg = av0[...] + av1[...]
    agg = jnp.where(sden > 0.0, agg / sden, 0.0)
    x = hvg + agg @ wo[...]
    mu = jnp.mean(x, axis=0, keepdims=True)
    xc = x - mu
    var = jnp.mean(xc * xc, axis=0, keepdims=True)
    x = xc * lax.rsqrt(var + 1e-5) * g0[...] + b0[...]
    f = f2b[...]
    for cix in range(4):
        hid = jnp.maximum(x @ f1w[:, cix * H:(cix + 1) * H]
                          + f1b[:, cix * H:(cix + 1) * H], 0.0)
        f = f + hid @ f2w[cix * H:(cix + 1) * H, :]
    y = x + f
    mu2 = jnp.mean(y, axis=0, keepdims=True)
    yc = y - mu2
    var2 = jnp.mean(yc * yc, axis=0, keepdims=True)
    h_pre = yc * lax.rsqrt(var2 + 1e-5) * g1[...] + b1[...]
    c = lax.dot_general(oh, h_pre, (((0,), (0,)), ((), ())))
    cnt = lax.dot_general(oh, jnp.ones((N, 1), jnp.float32),
                          (((0,), (0,)), ((), ())))
    c = c / jnp.maximum(cnt, 1.0)
    gg = jnp.maximum(c @ wg1[...] + bg1[...], 0.0)
    gg = jnp.maximum(gg @ wg2[...] + bg2[...], 0.0)
    gg = jax.nn.sigmoid(gg @ wg3[...] + bg3[...])
    return h_pre, gg, oh


def _node_body_mid(*refs):
    h_o, gg_o = refs[-2], refs[-1]
    h_pre, gg, _ = _node_body_common(*refs[:-2])
    h_o[...] = h_pre
    gg_o[...] = gg


def _node_body_last(*refs):
    wr, br, out = refs[-3], refs[-2], refs[-1]
    h_pre, gg, oh = _node_body_common(*refs[:-3])
    hv = h_pre * (oh @ gg)
    lgt = hv @ wr[...] + br[...]
    mx = jnp.max(lgt, axis=1, keepdims=True)
    sh = lgt - mx
    out[...] = sh - jnp.log(jnp.sum(jnp.exp(sh), axis=1, keepdims=True))


def _node(hv, ggp, av0, av1, as0, as1, bid2, p, readout):
    args = [hv, ggp, av0, av1, as0, as1, bid2,
            p['WO'], p['bn0'][0].reshape(1, H), p['bn0'][1].reshape(1, H),
            p['F1'][0], p['F1'][1].reshape(1, 4 * H),
            p['F2'][0], p['F2'][1].reshape(1, H),
            p['bn1'][0].reshape(1, H), p['bn1'][1].reshape(1, H),
            p['G1'][0], p['G1'][1].reshape(1, H),
            p['G2'][0], p['G2'][1].reshape(1, H),
            p['G3'][0], p['G3'][1].reshape(1, H)]
    if readout is None:
        body = _node_body_mid
        out_shape = [jax.ShapeDtypeStruct((N, H), jnp.float32),
                     jax.ShapeDtypeStruct((G, H), jnp.float32)]
    else:
        args += [readout[0], readout[1].reshape(1, 20)]
        body = _node_body_last
        out_shape = jax.ShapeDtypeStruct((N, 20), jnp.float32)
    return pl.pallas_call(body, out_shape=out_shape)(*args)


def _eupd_body(hs, hp, hd, s1, s2, gma, bta, we1s, we1p, we1d, be1,
               we2, be2, we3, be3, y_o, s1_o, s2_o):
    i = pl.program_id(0)
    mu = s1[0:1, :] * (1.0 / E)
    var = s2[0:1, :] * (1.0 / E) - mu * mu
    hp_ = (hp[...] - mu) * lax.rsqrt(var + 1e-5) * gma[...] + bta[...]
    m = jax.nn.gelu(_mm(hp_, we1p[...]) + _mm(hs[...], we1s[...])
                    + _mm(hd[...], we1d[...]) + be1[...])
    m = jax.nn.gelu(_mm(m, we2[...]) + be2[...])
    m = _mm(m, we3[...]) + be3[...]
    y = hp_ + m
    y_o[...] = y

    @pl.when(i == 0)
    def _():
        s1_o[...] = jnp.zeros((8, 128), jnp.float32)
        s2_o[...] = jnp.zeros((8, 128), jnp.float32)

    s1_o[...] = s1_o[...] + jnp.sum(y, axis=0, keepdims=True)
    s2_o[...] = s2_o[...] + jnp.sum(y * y, axis=0, keepdims=True)


def _edge_update(hs, hp, hd, hpstats, p):
    eb = lambda w: pl.BlockSpec((BE, w), lambda i: (i, 0))
    cb = lambda a: pl.BlockSpec(a.shape, lambda i: (0,) * a.ndim)
    we1s, we1p, we1d = p['E1'][0][:H], p['E1'][0][H:2 * H], p['E1'][0][2 * H:]
    s1, s2, gma, bta = hpstats
    args = (hs, hp, hd, s1, s2, gma, bta,
            we1s, we1p, we1d, p['E1'][1].reshape(1, H),
            p['E2'][0], p['E2'][1].reshape(1, H),
            p['E3'][0], p['E3'][1].reshape(1, H))
    acc = pl.BlockSpec((8, 128), lambda i: (0, 0))
    bb = pl.BlockSpec((1, H), lambda i: (0, 0))
    return pl.pallas_call(
        _eupd_body,
        grid=(NEB,),
        in_specs=[eb(H), eb(H), eb(H), acc, acc, bb, bb]
                 + [cb(a) for a in args[7:]],
        out_specs=[eb(H), acc, acc],
        out_shape=[jax.ShapeDtypeStruct((E, H), jnp.float32),
                   jax.ShapeDtypeStruct((8, 128), jnp.float32),
                   jax.ShapeDtypeStruct((8, 128), jnp.float32)],
    )(*args)


def kernel(h_V, h_P, P_idx, batch_id, params):
    idx3g = P_idx.reshape(NW, GIT, CHUNK)
    src3 = P_idx[0].reshape(NW, SIT, CHUNK)
    bid2 = batch_id.reshape(N, 1)
    bsrc = batch_id[P_idx[0]].reshape(E, 1)
    bdst = batch_id[P_idx[1]].reshape(E, 1)
    z_v = jnp.zeros((NPAD, H), jnp.float32)

    hp = h_P
    s1z = jnp.zeros((8, 128), jnp.float32)
    s2z = jnp.full((8, 128), float(E) * (1.0 - 1e-5), jnp.float32)
    hpstats = (s1z, s2z, jnp.ones((1, H), jnp.float32),
               jnp.zeros((1, H), jnp.float32))
    hv = h_V
    gg = jnp.ones((G, H), jnp.float32)
    g = _sc_gather(hv, idx3g)
    hs, hd = g[:E], g[E:]
    layers = params['layers']
    for l, p in enumerate(layers):
        lg, v, gm = _edge_attn(hs, hp, hd, hpstats, bsrc, bdst, gg, p)
        p_arr, ew = _edge_weight(lg, v, gm)
        acc_v = _sc_scatter_v(p_arr, src3, z_v)
        acc_s = _sc_scatter_v(ew, src3, z_v)
        last = l == len(layers) - 1
        res = _node(hv, gg, acc_v[:N], acc_v[NPAD:NPAD + N], acc_s[:N],
                    acc_s[NPAD:NPAD + N], bid2, p,
                    params['readout'] if last else None)
        if last:
            return res
        hv, gg = res
        g = _sc_gather(hv, idx3g)
        hs, hd = g[:E], g[E:]
        y, s1, s2 = _edge_update(hs, hp, hd, hpstats, p)
        hp = y
        hpstats = (s1, s2, p['bne'][0].reshape(1, H),
                   p['bne'][1].reshape(1, H))


# speedup vs baseline: 13.5090x; 1.0944x over previous
import functools

import jax
import jax.numpy as jnp
from jax import lax
from jax.experimental import pallas as pl
from jax.experimental.pallas import tpu as pltpu
from jax.experimental.pallas import tpu_sc as plsc

N = 10000
E = 320000
H = 128
HEADS = 4
DH = H // HEADS
G = 16
NC, NS = 2, 16
NW = NC * NS
CHUNK = 80
GIT = (2 * E) // NW // CHUNK
SIT = E // NW // CHUNK
EPW = E // NW
NPAD = 10240
NPT = NPAD // NS
BE = 1280
NEB = E // BE
ISD = float(DH) ** -0.5

_MESH = plsc.VectorSubcoreMesh(core_axis_name="c", subcore_axis_name="s")


@functools.partial(
    pl.kernel,
    out_type=jax.ShapeDtypeStruct((2 * E, H), jnp.float32),
    mesh=_MESH,
    scratch_types=[
        pltpu.VMEM((GIT, CHUNK), jnp.int32),
        pltpu.VMEM((2, CHUNK, H), jnp.float32),
        pltpu.SemaphoreType.DMA((2,)),
    ],
)
def _sc_gather(table, idx3, out, idx_v, buf, sem):
    wid = lax.axis_index("s") * NC + lax.axis_index("c")
    base = wid * (GIT * CHUNK)
    pltpu.sync_copy(idx3.at[wid], idx_v)
    pltpu.async_copy(table.at[idx_v.at[0]], buf.at[0], sem.at[0])

    def body(k, carry):
        slot = lax.rem(k, 2)
        nslot = lax.rem(k + 1, 2)

        @pl.when(k + 1 < GIT)
        def _():
            pltpu.async_copy(table.at[idx_v.at[k + 1]], buf.at[nslot],
                             sem.at[nslot])

        pltpu.make_async_copy(table.at[idx_v.at[k]], buf.at[slot],
                              sem.at[slot]).wait()
        pltpu.sync_copy(buf.at[slot], out.at[pl.ds(base + k * CHUNK, CHUNK)])
        return carry

    lax.fori_loop(0, GIT, body, 0)


def _make_sc_scatter(w):
    @functools.partial(
        pl.kernel,
        out_type=jax.ShapeDtypeStruct((NC * NPAD, w), jnp.float32),
        mesh=_MESH,
        scratch_types=[
            pltpu.VMEM((SIT, CHUNK), jnp.int32),
            pltpu.VMEM((2, CHUNK, w), jnp.float32),
            pltpu.VMEM_SHARED((NPAD, w), jnp.float32),
            pltpu.SemaphoreType.DMA((2,)),
        ],
    )
    def scat(vals, src3, z_v, out, idx_v, vbuf, acc, sem):
        cid = lax.axis_index("c")
        sid = lax.axis_index("s")
        wid = sid * NC + cid
        pltpu.sync_copy(z_v.at[pl.ds(sid * NPT, NPT)],
                        acc.at[pl.ds(sid * NPT, NPT)])
        plsc.subcore_barrier()

        base = wid * EPW
        pltpu.sync_copy(src3.at[wid], idx_v)
        pltpu.async_copy(vals.at[pl.ds(base, CHUNK)], vbuf.at[0], sem.at[0])

        def body(k, carry):
            slot = lax.rem(k, 2)
            nslot = lax.rem(k + 1, 2)

            @pl.when(k + 1 < SIT)
            def _():
                pltpu.async_copy(vals.at[pl.ds(base + (k + 1) * CHUNK, CHUNK)],
                                 vbuf.at[nslot], sem.at[nslot])

            pltpu.make_async_copy(vals.at[pl.ds(base + k * CHUNK, CHUNK)],
                                  vbuf.at[slot], sem.at[slot]).wait()
            pltpu.sync_copy(vbuf.at[slot], acc.at[idx_v.at[k]], add=True)
            return carry

        lax.fori_loop(0, SIT, body, 0)
        plsc.subcore_barrier()
        pltpu.sync_copy(acc.at[pl.ds(sid * NPT, NPT)],
                        out.at[pl.ds(cid * NPAD + sid * NPT, NPT)])

    return scat


_sc_scatter_v = _make_sc_scatter(H)


def _mm(a, b):
    return lax.dot_general(a.astype(jnp.bfloat16), b.astype(jnp.bfloat16),
                           (((1,), (0,)), ((), ())),
                           preferred_element_type=jnp.float32)


def _head_expand():
    lane_head = lax.broadcasted_iota(jnp.int32, (HEADS, H), 1) // DH
    row = lax.broadcasted_iota(jnp.int32, (HEADS, H), 0)
    return (lane_head == row).astype(jnp.float32)


def _graph_onehot(ids, rows):
    return (ids[...] == lax.broadcasted_iota(jnp.int32, (rows, G), 1)
            ).astype(jnp.float32)


def _attn_body(hs, hp, hd, s1, s2, gma, bta, bs, bd, gg, wb1s, wb1p, wb1d,
               bb1, wb2, bb2, wb3, bb3, wv1p, wv1d, bv1, wv2, bv2, wv3, bv3,
               lg_o, v_o, gm_o):
    i = pl.program_id(0)
    gg_ = gg[...]
    hs_ = hs[...] * (_graph_onehot(bs, BE) @ gg_)
    hd_ = hd[...] * (_graph_onehot(bd, BE) @ gg_)
    mu = s1[0:1, :] * (1.0 / E)
    var = s2[0:1, :] * (1.0 / E) - mu * mu
    hp_ = (hp[...] - mu) * lax.rsqrt(var + 1e-5) * gma[...] + bta[...]
    t = jax.nn.gelu(_mm(hs_, wb1s[...]) + _mm(hp_, wb1p[...])
                    + _mm(hd_, wb1d[...]) + bb1[...])
    t = jax.nn.gelu(_mm(t, wb2[...]) + bb2[...])
    lg = _mm(t, wb3[...]) + bb3[...]
    u = jax.nn.gelu(_mm(hp_, wv1p[...]) + _mm(hd_, wv1d[...]) + bv1[...])
    u = jax.nn.gelu(_mm(u, wv2[...]) + bv2[...])
    v = _mm(u, wv3[...]) + bv3[...]
    lg_o[...] = lg
    v_o[...] = v
    m = jnp.max(lg)

    @pl.when(i == 0)
    def _():
        gm_o[...] = jnp.full((8, 128), -3e38, jnp.float32)

    gm_o[...] = jnp.maximum(gm_o[...], m)


def _edge_attn(hs, hp, hd, hpstats, bsrc, bdst, gg, p):
    eb = lambda w: pl.BlockSpec((BE, w), lambda i: (i, 0))
    cb = lambda a: pl.BlockSpec(a.shape, lambda i: (0,) * a.ndim)
    wb1s, wb1p, wb1d = p['B1'][0][:H], p['B1'][0][H:2 * H], p['B1'][0][2 * H:]
    wv1p, wv1d = p['V1'][0][:H], p['V1'][0][H:]
    s1, s2, gma, bta = hpstats
    args = (hs, hp, hd, s1, s2, gma, bta, bsrc, bdst, gg,
            wb1s, wb1p, wb1d, p['B1'][1].reshape(1, H),
            p['B2'][0], p['B2'][1].reshape(1, H),
            p['B3'][0], p['B3'][1].reshape(1, HEADS),
            wv1p, wv1d, p['V1'][1].reshape(1, H),
            p['V2'][0], p['V2'][1].reshape(1, H),
            p['V3'][0], p['V3'][1].reshape(1, H))
    sb = pl.BlockSpec((8, 128), lambda i: (0, 0))
    bb = pl.BlockSpec((1, H), lambda i: (0, 0))
    return pl.pallas_call(
        _attn_body,
        grid=(NEB,),
        in_specs=[eb(H), eb(H), eb(H), sb, sb, bb, bb, eb(1), eb(1), cb(gg)]
                 + [cb(a) for a in args[10:]],
        out_specs=[eb(HEADS), eb(H), pl.BlockSpec((8, 128), lambda i: (0, 0))],
        out_shape=[jax.ShapeDtypeStruct((E, HEADS), jnp.float32),
                   jax.ShapeDtypeStruct((E, H), jnp.float32),
                   jax.ShapeDtypeStruct((8, 128), jnp.float32)],
    )(*args)


def _weight_body(lg, v, gm, p_o, e_o):
    r4 = _head_expand()
    eb = jnp.exp((lg[...] - gm[0:1, 0:1]) * ISD) @ r4
    e_o[...] = eb
    p_o[...] = eb * v[...]


def _edge_weight(lg, v, gm):
    eb = lambda w: pl.BlockSpec((BE, w), lambda i: (i, 0))
    return pl.pallas_call(
        _weight_body,
        grid=(NEB,),
        in_specs=[eb(HEADS), eb(H), pl.BlockSpec((8, 128), lambda i: (0, 0))],
        out_specs=[eb(H), eb(H)],
        out_shape=[jax.ShapeDtypeStruct((E, H), jnp.float32),
                   jax.ShapeDtypeStruct((E, H), jnp.float32)],
    )(lg, v, gm)


def _node_body_common(hv, ggp, av0, av1, as0, as1, bid, wo, g0, b0, f1w, f1b,
                      f2w, f2b, g1, b1, wg1, bg1, wg2, bg2, wg3, bg3):
    oh = _graph_onehot(bid, N)
    hvg = hv[...] * (oh @ ggp[...])
    sden = as0[...] + as1[...]
    agg = av0[...] + av1[...]
    agg = jnp.where(sden > 0.0, agg / sden, 0.0)
    x = hvg + agg @ wo[...]
    mu = jnp.mean(x, axis=0, keepdims=True)
    xc = x - mu
    var = jnp.mean(xc * xc, axis=0, keepdims=True)
    x = xc * lax.rsqrt(var + 1e-5) * g0[...] + b0[...]
    f = f2b[...]
    for cix in range(4):
        hid = jnp.maximum(x @ f1w[:, cix * H:(cix + 1) * H]
                          + f1b[:, cix * H:(cix + 1) * H], 0.0)
        f = f + hid @ f2w[cix * H:(cix + 1) * H, :]
    y = x + f
    mu2 = jnp.mean(y, axis=0, keepdims=True)
    yc = y - mu2
    var2 = jnp.mean(yc * yc, axis=0, keepdims=True)
    h_pre = yc * lax.rsqrt(var2 + 1e-5) * g1[...] + b1[...]
    c = lax.dot_general(oh, h_pre, (((0,), (0,)), ((), ())))
    cnt = lax.dot_general(oh, jnp.ones((N, 1), jnp.float32),
                          (((0,), (0,)), ((), ())))
    c = c / jnp.maximum(cnt, 1.0)
    gg = jnp.maximum(c @ wg1[...] + bg1[...], 0.0)
    gg = jnp.maximum(gg @ wg2[...] + bg2[...], 0.0)
    gg = jax.nn.sigmoid(gg @ wg3[...] + bg3[...])
    return h_pre, gg, oh


def _node_body_mid(*refs):
    h_o, gg_o = refs[-2], refs[-1]
    h_pre, gg, _ = _node_body_common(*refs[:-2])
    h_o[...] = h_pre
    gg_o[...] = gg


def _node_body_last(*refs):
    wr, br, out = refs[-3], refs[-2], refs[-1]
    h_pre, gg, oh = _node_body_common(*refs[:-3])
    hv = h_pre * (oh @ gg)
    lgt = hv @ wr[...] + br[...]
    mx = jnp.max(lgt, axis=1, keepdims=True)
    sh = lgt - mx
    out[...] = sh - jnp.log(jnp.sum(jnp.exp(sh), axis=1, keepdims=True))


def _node(hv, ggp, av0, av1, as0, as1, bid2, p, readout):
    args = [hv, ggp, av0, av1, as0, as1, bid2,
            p['WO'], p['bn0'][0].reshape(1, H), p['bn0'][1].reshape(1, H),
            p['F1'][0], p['F1'][1].reshape(1, 4 * H),
            p['F2'][0], p['F2'][1].reshape(1, H),
            p['bn1'][0].reshape(1, H), p['bn1'][1].reshape(1, H),
            p['G1'][0], p['G1'][1].reshape(1, H),
            p['G2'][0], p['G2'][1].reshape(1, H),
            p['G3'][0], p['G3'][1].reshape(1, H)]
    if readout is None:
        body = _node_body_mid
        out_shape = [jax.ShapeDtypeStruct((N, H), jnp.float32),
                     jax.ShapeDtypeStruct((G, H), jnp.float32)]
    else:
        args += [readout[0], readout[1].reshape(1, 20)]
        body = _node_body_last
        out_shape = jax.ShapeDtypeStruct((N, 20), jnp.float32)
    return pl.pallas_call(body, out_shape=out_shape)(*args)


def _eupd_body(hs, hp, hd, s1, s2, gma, bta, we1s, we1p, we1d, be1,
               we2, be2, we3, be3, y_o, s1_o, s2_o):
    i = pl.program_id(0)
    mu = s1[0:1, :] * (1.0 / E)
    var = s2[0:1, :] * (1.0 / E) - mu * mu
    hp_ = (hp[...] - mu) * lax.rsqrt(var + 1e-5) * gma[...] + bta[...]
    m = jax.nn.gelu(_mm(hp_, we1p[...]) + _mm(hs[...], we1s[...])
                    + _mm(hd[...], we1d[...]) + be1[...])
    m = jax.nn.gelu(_mm(m, we2[...]) + be2[...])
    m = _mm(m, we3[...]) + be3[...]
    y = hp_ + m
    y_o[...] = y

    @pl.when(i == 0)
    def _():
        s1_o[...] = jnp.zeros((8, 128), jnp.float32)
        s2_o[...] = jnp.zeros((8, 128), jnp.float32)

    s1_o[...] = s1_o[...] + jnp.sum(y, axis=0, keepdims=True)
    s2_o[...] = s2_o[...] + jnp.sum(y * y, axis=0, keepdims=True)


def _edge_update(hs, hp, hd, hpstats, p):
    eb = lambda w: pl.BlockSpec((BE, w), lambda i: (i, 0))
    cb = lambda a: pl.BlockSpec(a.shape, lambda i: (0,) * a.ndim)
    we1s, we1p, we1d = p['E1'][0][:H], p['E1'][0][H:2 * H], p['E1'][0][2 * H:]
    s1, s2, gma, bta = hpstats
    args = (hs, hp, hd, s1, s2, gma, bta,
            we1s, we1p, we1d, p['E1'][1].reshape(1, H),
            p['E2'][0], p['E2'][1].reshape(1, H),
            p['E3'][0], p['E3'][1].reshape(1, H))
    acc = pl.BlockSpec((8, 128), lambda i: (0, 0))
    bb = pl.BlockSpec((1, H), lambda i: (0, 0))
    return pl.pallas_call(
        _eupd_body,
        grid=(NEB,),
        in_specs=[eb(H), eb(H), eb(H), acc, acc, bb, bb]
                 + [cb(a) for a in args[7:]],
        out_specs=[eb(H), acc, acc],
        out_shape=[jax.ShapeDtypeStruct((E, H), jnp.float32),
                   jax.ShapeDtypeStruct((8, 128), jnp.float32),
                   jax.ShapeDtypeStruct((8, 128), jnp.float32)],
    )(*args)


def kernel(h_V, h_P, P_idx, batch_id, params):
    idx3g = P_idx.reshape(NW, GIT, CHUNK)
    src3 = P_idx[0].reshape(NW, SIT, CHUNK)
    bid2 = batch_id.reshape(N, 1)
    bsrc = batch_id[P_idx[0]].reshape(E, 1)
    bdst = batch_id[P_idx[1]].reshape(E, 1)
    z_v = jnp.zeros((NPAD, H), jnp.float32)

    hp = h_P
    s1z = jnp.zeros((8, 128), jnp.float32)
    s2z = jnp.full((8, 128), float(E) * (1.0 - 1e-5), jnp.float32)
    hpstats = (s1z, s2z, jnp.ones((1, H), jnp.float32),
               jnp.zeros((1, H), jnp.float32))
    hv = h_V
    gg = jnp.ones((G, H), jnp.float32)
    g = _sc_gather(hv, idx3g)
    hs, hd = g[:E], g[E:]
    layers = params['layers']
    for l, p in enumerate(layers):
        lg, v, gm = _edge_attn(hs, hp, hd, hpstats, bsrc, bdst, gg, p)
        p_arr, ew = _edge_weight(lg, v, gm)
        acc_v = _sc_scatter_v(p_arr, src3, z_v)
        acc_s = _sc_scatter_v(ew, src3, z_v)
        last = l == len(layers) - 1
        res = _node(hv, gg, acc_v[:N], acc_v[NPAD:NPAD + N], acc_s[:N],
                    acc_s[NPAD:NPAD + N], bid2, p,
                    params['readout'] if last else None)
        if last:
            return res
        hv, gg = res
        g = _sc_gather(hv, idx3g)
        hs, hd = g[:E], g[E:]
        y, s1, s2 = _edge_update(hs, hp, hd, hpstats, p)
        hp = y
        hpstats = (s1, s2, p['bne'][0].reshape(1, H),
                   p['bne'][1].reshape(1, H))
